# baseline (speedup 1.0000x reference)
import functools

import jax
import jax.numpy as jnp
from jax import lax
from jax.experimental import pallas as pl
from jax.experimental.pallas import tpu as pltpu
from jax.experimental.pallas import tpu_sc as plsc

NUM_FIELDS = 26
VOCAB = 100000
EMBED_DIM = 32
BATCH = 16384

NUM_LINES = NUM_FIELDS * EMBED_DIM
NUM_WORKERS = 32
LINES_PER_W = NUM_LINES // NUM_WORKERS
LANES = 16
CHUNK = 4096
NCHUNK = BATCH // CHUNK


def _body(cat_hbm, tab_hbm, col_hbm, out_hbm, line_v, idx_v, out_v, col_v,
          line_sem, out_sem0, out_sem1):
    nc = 2
    wid = lax.axis_index("s") * nc + lax.axis_index("c")
    base = wid * LINES_PER_W
    out_sems = (out_sem0, out_sem1)

    pltpu.sync_copy(col_hbm, col_v)

    f0 = base // EMBED_DIM
    pltpu.sync_copy(cat_hbm.at[f0], idx_v)
    pltpu.async_copy(tab_hbm.at[base], line_v, line_sem)

    zeros16 = jnp.full((LANES,), 0, jnp.int32)

    def line_body(li, prev_f):
        l = base + li
        f = l // EMBED_DIM
        d = l % EMBED_DIM

        @pl.when(f != prev_f)
        def _():
            pltpu.sync_copy(cat_hbm.at[f], idx_v)

        bias = plsc.load_gather(col_v, [zeros16 + f, zeros16 + d])

        pltpu.make_async_copy(tab_hbm.at[l], line_v, line_sem).wait()

        for c in range(NCHUNK):
            slot = c % 2
            sem = out_sems[slot]

            if c >= 2:
                pltpu.make_async_copy(
                    out_v.at[slot], out_hbm.at[l, pl.ds(c * CHUNK, CHUNK)], sem
                ).wait()
            else:
                @pl.when(li > 0)
                def _():
                    pltpu.make_async_copy(
                        out_v.at[slot], out_hbm.at[l, pl.ds(c * CHUNK, CHUNK)],
                        sem,
                    ).wait()

            def gather_body(i, c3):
                s = pl.ds(c * CHUNK + i * LANES, LANES)
                vals = plsc.load_gather(line_v, [idx_v[s]])
                out_v[slot, pl.ds(i * LANES, LANES)] = vals + bias
                return c3

            lax.fori_loop(0, CHUNK // LANES, gather_body, 0, unroll=8)
            pltpu.async_copy(
                out_v.at[slot], out_hbm.at[l, pl.ds(c * CHUNK, CHUNK)], sem
            )

        @pl.when(li < LINES_PER_W - 1)
        def _():
            pltpu.async_copy(tab_hbm.at[l + 1], line_v, line_sem)

        return f

    lax.fori_loop(0, LINES_PER_W, line_body, f0)

    last = base + LINES_PER_W - 1
    for slot in range(2):
        pltpu.make_async_copy(
            out_v.at[slot], out_hbm.at[last, pl.ds(slot * CHUNK, CHUNK)],
            out_sems[slot],
        ).wait()


@jax.jit
def _run(cat_t, tab_t, col):
    mesh = plsc.VectorSubcoreMesh(core_axis_name="c", subcore_axis_name="s")
    k = functools.partial(
        pl.kernel,
        mesh=mesh,
        out_type=jax.ShapeDtypeStruct((NUM_LINES, BATCH), jnp.float32),
        scratch_types=[
            pltpu.VMEM((VOCAB,), jnp.float32),
            pltpu.VMEM((BATCH,), jnp.int32),
            pltpu.VMEM((2, CHUNK), jnp.float32),
            pltpu.VMEM((NUM_FIELDS, EMBED_DIM), jnp.float32),
            pltpu.SemaphoreType.DMA,
            pltpu.SemaphoreType.DMA,
            pltpu.SemaphoreType.DMA,
        ],
        compiler_params=pltpu.CompilerParams(
            use_tc_tiling_on_sc=True, needs_layout_passes=False
        ),
    )(_body)
    return k(cat_t, tab_t, col)


def kernel(categorical_inputs, tables, column_embedding):
    cat_t = categorical_inputs.astype(jnp.int32).T
    tab_t = tables.transpose(0, 2, 1).reshape(NUM_LINES, VOCAB)
    out_t = _run(cat_t, tab_t, column_embedding)
    return out_t.reshape(NUM_FIELDS, EMBED_DIM, BATCH).transpose(2, 0, 1)


# speedup vs baseline: 2.1124x; 2.1124x over previous
import functools

import jax
import jax.numpy as jnp
from jax import lax
from jax.experimental import pallas as pl
from jax.experimental.pallas import tpu as pltpu
from jax.experimental.pallas import tpu_sc as plsc

NUM_FIELDS = 26
VOCAB = 100000
EMBED_DIM = 32
BATCH = 16384

NUM_LINES = NUM_FIELDS * EMBED_DIM
NUM_WORKERS = 32
LINES_PER_W = NUM_LINES // NUM_WORKERS
LANES = 16
CHUNK = 4096
NCHUNK = BATCH // CHUNK


def _body(cat_hbm, tab_hbm, col_hbm, out_hbm, line_v, idx_v, out_v, col_v,
          line_sem, out_sem0, out_sem1):
    nc = 2
    wid = lax.axis_index("s") * nc + lax.axis_index("c")
    base = wid * LINES_PER_W
    out_sems = (out_sem0, out_sem1)

    pltpu.sync_copy(col_hbm, col_v)

    f0 = base // EMBED_DIM
    pltpu.sync_copy(cat_hbm.at[f0], idx_v)
    pltpu.async_copy(tab_hbm.at[base], line_v, line_sem)

    zeros16 = jnp.full((LANES,), 0, jnp.int32)

    def line_body(li, prev_f):
        l = base + li
        f = l // EMBED_DIM
        d = l % EMBED_DIM

        @pl.when(f != prev_f)
        def _():
            pltpu.sync_copy(cat_hbm.at[f], idx_v)

        bias = plsc.load_gather(col_v, [zeros16 + f, zeros16 + d])

        pltpu.make_async_copy(tab_hbm.at[l], line_v, line_sem).wait()

        for c in range(NCHUNK):
            slot = c % 2
            sem = out_sems[slot]

            if c >= 2:
                pltpu.make_async_copy(
                    out_v.at[slot], out_hbm.at[l, pl.ds(c * CHUNK, CHUNK)], sem
                ).wait()
            else:
                @pl.when(li > 0)
                def _():
                    pltpu.make_async_copy(
                        out_v.at[slot], out_hbm.at[l, pl.ds(c * CHUNK, CHUNK)],
                        sem,
                    ).wait()

            @plsc.parallel_loop(0, CHUNK, LANES, unroll=8)
            def _(i):
                vals = plsc.load_gather(line_v, [idx_v[pl.ds(c * CHUNK + i, LANES)]])
                out_v[slot, pl.ds(i, LANES)] = vals + bias
            pltpu.async_copy(
                out_v.at[slot], out_hbm.at[l, pl.ds(c * CHUNK, CHUNK)], sem
            )

        @pl.when(li < LINES_PER_W - 1)
        def _():
            pltpu.async_copy(tab_hbm.at[l + 1], line_v, line_sem)

        return f

    lax.fori_loop(0, LINES_PER_W, line_body, f0)

    last = base + LINES_PER_W - 1
    for slot in range(2):
        pltpu.make_async_copy(
            out_v.at[slot], out_hbm.at[last, pl.ds(slot * CHUNK, CHUNK)],
            out_sems[slot],
        ).wait()


@jax.jit
def _run(cat_t, tab_t, col):
    mesh = plsc.VectorSubcoreMesh(core_axis_name="c", subcore_axis_name="s")
    k = functools.partial(
        pl.kernel,
        mesh=mesh,
        out_type=jax.ShapeDtypeStruct((NUM_LINES, BATCH), jnp.float32),
        scratch_types=[
            pltpu.VMEM((VOCAB,), jnp.float32),
            pltpu.VMEM((BATCH,), jnp.int32),
            pltpu.VMEM((2, CHUNK), jnp.float32),
            pltpu.VMEM((NUM_FIELDS, EMBED_DIM), jnp.float32),
            pltpu.SemaphoreType.DMA,
            pltpu.SemaphoreType.DMA,
            pltpu.SemaphoreType.DMA,
        ],
        compiler_params=pltpu.CompilerParams(
            use_tc_tiling_on_sc=True, needs_layout_passes=False
        ),
    )(_body)
    return k(cat_t, tab_t, col)


def kernel(categorical_inputs, tables, column_embedding):
    cat_t = categorical_inputs.astype(jnp.int32).T
    tab_t = tables.transpose(0, 2, 1).reshape(NUM_LINES, VOCAB)
    out_t = _run(cat_t, tab_t, column_embedding)
    return out_t.reshape(NUM_FIELDS, EMBED_DIM, BATCH).transpose(2, 0, 1)
